# baseline (speedup 1.0000x reference)
import functools

import jax
import jax.numpy as jnp
import numpy as np
from jax import lax
from jax.experimental import pallas as pl
from jax.experimental.pallas import tpu as pltpu
from jax.experimental.pallas import tpu_sc as plsc

_K = 8

_NC = 2
_NS = 16
_NW = _NC * _NS
_L = 16
_R = 8
_NB = 4

_LOG2P = (
    0.0008254628229340533, 1.415653190432736, -0.5687040530057521,
    0.15270028479752185,
)
_LN2 = 0.6931471805599453
_LOG2E = 1.4426950408889634


def _vlog2(y):
    bits = lax.bitcast_convert_type(y, jnp.int32)
    e = (bits >> 23) - 127
    f = lax.bitcast_convert_type((bits & 0x007FFFFF) | 0x3F800000, jnp.float32)
    t = f - 1.0
    p = _LOG2P[-1] * t + _LOG2P[-2]
    for c in _LOG2P[-3::-1]:
        p = p * t + c
    return e.astype(jnp.float32) + p


def _mix_logprob2(xv, mk, nk, ck):
    ls = [(xv - mk[k]) * (xv - mk[k]) * nk[k] + ck[k] for k in range(_K)]
    m0 = jnp.maximum(jnp.maximum(ls[0], ls[1]), jnp.maximum(ls[2], ls[3]))
    m1 = jnp.maximum(jnp.maximum(ls[4], ls[5]), jnp.maximum(ls[6], ls[7]))
    lmax = jnp.maximum(m0, m1)
    s = None
    for k in range(_K):
        e = jnp.exp(ls[k] - lmax)
        s = e if s is None else s + e
    return lmax, s


def _sc_body(coef_hbm, x_hbm, o_hbm, coef_v, buf, isem, osem):
    wid = lax.axis_index("s") * _NC + lax.axis_index("c")
    rows_per = x_hbm.shape[0] // _NW
    z = x_hbm.shape[1]
    nchunk = rows_per // _R
    base = wid * rows_per
    pltpu.sync_copy(coef_hbm, coef_v)

    def in_cp(c):
        p = lax.bitwise_and(c, _NB - 1)
        return pltpu.make_async_copy(
            x_hbm.at[pl.ds(base + c * _R, _R)], buf.at[p], isem.at[p])

    def out_cp(c):
        p = lax.bitwise_and(c, _NB - 1)
        return pltpu.make_async_copy(
            buf.at[p], o_hbm.at[pl.ds(base + c * _R, _R)], osem.at[p])

    in_cp(0).start()

    def chunk_body(c, carry):
        p = lax.bitwise_and(c, _NB - 1)

        @pl.when(c >= _NB - 1)
        def _():
            out_cp(c - (_NB - 1)).wait()

        @pl.when(c + 1 < nchunk)
        def _():
            in_cp(c + 1).start()

        in_cp(c).wait()

        def z_body(zb, c2):
            zsl = pl.ds(zb * _L, _L)
            mk = [coef_v[k, zsl] for k in range(_K)]
            nk = [coef_v[_K + k, zsl] for k in range(_K)]
            ck = [coef_v[2 * _K + k, zsl] for k in range(_K)]
            for r in range(_R):
                lmax, s = _mix_logprob2(buf[p, r, zsl], mk, nk, ck)
                buf[p, r, zsl] = lmax + _vlog2(s) * _LN2
            return c2

        lax.fori_loop(0, z // _L, z_body, 0)
        out_cp(c).start()
        return carry

    lax.fori_loop(0, nchunk, chunk_body, 0)
    for c in range(nchunk - _NB + 1, nchunk):
        out_cp(jnp.int32(c)).wait()


def _sc_call(coef, x):
    b, z = x.shape
    mesh = plsc.VectorSubcoreMesh(core_axis_name="c", subcore_axis_name="s")
    return pl.kernel(
        _sc_body,
        mesh=mesh,
        out_type=jax.ShapeDtypeStruct((b, z), jnp.float32),
        scratch_types=[
            pltpu.VMEM((3 * _K, z), jnp.float32),
            pltpu.VMEM((_NB, _R, z), jnp.float32),
        ] + [pltpu.SemaphoreType.DMA((_NB,)), pltpu.SemaphoreType.DMA((_NB,))],
    )(coef, x)


@jax.jit
def kernel(x, mean_list, scale_list, weight_logits):
    B, Z = x.shape
    scale = jax.nn.softplus(scale_list)
    ninv = -0.5 / (scale * scale)
    log_w = jax.nn.log_softmax(weight_logits, axis=-1)
    cns = (-jnp.log(scale) - 0.5 * np.log(2.0 * np.pi)
           + log_w[:, None, :])
    coef = jnp.concatenate(
        [mean_list[0].T, ninv[0].T, cns[0].T], axis=0)
    return _sc_call(coef, x)


# speedup vs baseline: 1.0426x; 1.0426x over previous
import jax
import jax.numpy as jnp
import numpy as np
from jax import lax
from jax.experimental import pallas as pl
from jax.experimental.pallas import tpu as pltpu
from jax.experimental.pallas import tpu_sc as plsc

_K = 8

_NC = 2
_NS = 16
_NW = _NC * _NS
_L = 16
_R = 16
_NB = 2

_LOG2P = (
    0.0008254628229340533, 1.415653190432736, -0.5687040530057521,
    0.15270028479752185,
)
_LN2 = 0.6931471805599453


def _vlog2(y):
    bits = lax.bitcast_convert_type(y, jnp.int32)
    e = (bits >> 23) - 127
    f = lax.bitcast_convert_type((bits & 0x007FFFFF) | 0x3F800000, jnp.float32)
    t = f - 1.0
    p = _LOG2P[-1] * t + _LOG2P[-2]
    for c in _LOG2P[-3::-1]:
        p = p * t + c
    return e.astype(jnp.float32) + p


def _mix_logprob(xv, mk, nk, ck):
    ls = [(xv - mk[k]) * (xv - mk[k]) * nk[k] + ck[k] for k in range(_K)]
    m0 = jnp.maximum(jnp.maximum(ls[0], ls[1]), jnp.maximum(ls[2], ls[3]))
    m1 = jnp.maximum(jnp.maximum(ls[4], ls[5]), jnp.maximum(ls[6], ls[7]))
    lmax = jnp.maximum(m0, m1)
    s = None
    for k in range(_K):
        e = jnp.exp(ls[k] - lmax)
        s = e if s is None else s + e
    return lmax, s


def _sc_body(coef_hbm, x_hbm, o_hbm, coef_v, buf, isem, osem):
    wid = lax.axis_index("s") * _NC + lax.axis_index("c")
    rows_per = x_hbm.shape[0] // _NW
    z = x_hbm.shape[1]
    nchunk = rows_per // _R
    base = wid * rows_per
    pltpu.sync_copy(coef_hbm, coef_v)

    def in_cp(c):
        p = lax.bitwise_and(c, _NB - 1)
        return pltpu.make_async_copy(
            x_hbm.at[pl.ds(base + c * _R, _R)], buf.at[p], isem.at[p])

    def out_cp(c):
        p = lax.bitwise_and(c, _NB - 1)
        return pltpu.make_async_copy(
            buf.at[p], o_hbm.at[pl.ds(base + c * _R, _R)], osem.at[p])

    in_cp(0).start()

    def chunk_body(c, carry):
        p = lax.bitwise_and(c, _NB - 1)

        @pl.when(c >= _NB - 1)
        def _():
            out_cp(c - (_NB - 1)).wait()

        @pl.when(c + 1 < nchunk)
        def _():
            in_cp(c + 1).start()

        in_cp(c).wait()

        def z_body(zb, c2):
            zsl = pl.ds(zb * _L, _L)
            mk = [coef_v[k, zsl] for k in range(_K)]
            nk = [coef_v[_K + k, zsl] for k in range(_K)]
            ck = [coef_v[2 * _K + k, zsl] for k in range(_K)]
            for r in range(_R):
                lmax, s = _mix_logprob(buf[p, r, zsl], mk, nk, ck)
                buf[p, r, zsl] = lmax + _vlog2(s) * _LN2
            return c2

        lax.fori_loop(0, z // _L, z_body, 0)
        out_cp(c).start()
        return carry

    lax.fori_loop(0, nchunk, chunk_body, 0)
    for c in range(nchunk - _NB + 1, nchunk):
        out_cp(jnp.int32(c)).wait()


def _sc_call(coef, x):
    b, z = x.shape
    mesh = plsc.VectorSubcoreMesh(core_axis_name="c", subcore_axis_name="s")
    return pl.kernel(
        _sc_body,
        mesh=mesh,
        out_type=jax.ShapeDtypeStruct((b, z), jnp.float32),
        scratch_types=[
            pltpu.VMEM((3 * _K, z), jnp.float32),
            pltpu.VMEM((_NB, _R, z), jnp.float32),
        ] + [pltpu.SemaphoreType.DMA((_NB,)), pltpu.SemaphoreType.DMA((_NB,))],
    )(coef, x)


@jax.jit
def kernel(x, mean_list, scale_list, weight_logits):
    scale = jax.nn.softplus(scale_list)
    ninv = -0.5 / (scale * scale)
    log_w = jax.nn.log_softmax(weight_logits, axis=-1)
    cns = -jnp.log(scale) - 0.5 * np.log(2.0 * np.pi) + log_w[:, None, :]
    coef = jnp.concatenate(
        [mean_list[0].T, ninv[0].T, cns[0].T], axis=0)
    return _sc_call(coef, x)


# speedup vs baseline: 1.0490x; 1.0061x over previous
import jax
import jax.numpy as jnp
import numpy as np
from jax import lax
from jax.experimental import pallas as pl
from jax.experimental.pallas import tpu as pltpu
from jax.experimental.pallas import tpu_sc as plsc

_K = 8

_NC = 2
_NS = 16
_NW = _NC * _NS
_L = 16
_R = 16
_NB = 2

_LOG2P = (
    0.0008254628229340533, 1.415653190432736, -0.5687040530057521,
    0.15270028479752185,
)
_LN2 = 0.6931471805599453


def _vlog2(y):
    bits = lax.bitcast_convert_type(y, jnp.int32)
    e = (bits >> 23) - 127
    f = lax.bitcast_convert_type((bits & 0x007FFFFF) | 0x3F800000, jnp.float32)
    t = f - 1.0
    p = _LOG2P[-1] * t + _LOG2P[-2]
    for c in _LOG2P[-3::-1]:
        p = p * t + c
    return e.astype(jnp.float32) + p


def _mix_logprob(xv, mk, nk, ck):
    ls = [(xv - mk[k]) * (xv - mk[k]) * nk[k] + ck[k] for k in range(_K)]
    m0 = jnp.maximum(jnp.maximum(ls[0], ls[1]), jnp.maximum(ls[2], ls[3]))
    m1 = jnp.maximum(jnp.maximum(ls[4], ls[5]), jnp.maximum(ls[6], ls[7]))
    lmax = jnp.maximum(m0, m1)
    s = None
    for k in range(_K):
        e = jnp.exp(ls[k] - lmax)
        s = e if s is None else s + e
    return lmax, s


def _sc_body(coef_hbm, x_hbm, o_hbm, coef_v, buf, isem, osem):
    wid = lax.axis_index("s") * _NC + lax.axis_index("c")
    rows_per = x_hbm.shape[0] // _NW
    z = x_hbm.shape[1]
    nchunk = rows_per // _R
    base = wid * rows_per

    def in_cp(c):
        p = lax.bitwise_and(c, _NB - 1)
        return pltpu.make_async_copy(
            x_hbm.at[pl.ds(base + c * _R, _R)], buf.at[p], isem.at[p])

    def out_cp(c):
        p = lax.bitwise_and(c, _NB - 1)
        return pltpu.make_async_copy(
            buf.at[p], o_hbm.at[pl.ds(base + c * _R, _R)], osem.at[p])

    in_cp(0).start()
    pltpu.sync_copy(coef_hbm, coef_v)

    def chunk_body(c, carry):
        p = lax.bitwise_and(c, _NB - 1)

        @pl.when(c >= _NB - 1)
        def _():
            out_cp(c - (_NB - 1)).wait()

        @pl.when(c + 1 < nchunk)
        def _():
            in_cp(c + 1).start()

        in_cp(c).wait()

        def z_body(zb, c2):
            zsl = pl.ds(zb * _L, _L)
            mk = [coef_v[k, zsl] for k in range(_K)]
            nk = [coef_v[_K + k, zsl] for k in range(_K)]
            ck = [coef_v[2 * _K + k, zsl] for k in range(_K)]
            for r in range(_R):
                lmax, s = _mix_logprob(buf[p, r, zsl], mk, nk, ck)
                buf[p, r, zsl] = lmax + _vlog2(s) * _LN2
            return c2

        lax.fori_loop(0, z // _L, z_body, 0)
        out_cp(c).start()
        return carry

    lax.fori_loop(0, nchunk, chunk_body, 0)
    for c in range(nchunk - _NB + 1, nchunk):
        out_cp(jnp.int32(c)).wait()


def _sc_call(coef, x):
    b, z = x.shape
    mesh = plsc.VectorSubcoreMesh(core_axis_name="c", subcore_axis_name="s")
    return pl.kernel(
        _sc_body,
        mesh=mesh,
        out_type=jax.ShapeDtypeStruct((b, z), jnp.float32),
        scratch_types=[
            pltpu.VMEM((3 * _K, z), jnp.float32),
            pltpu.VMEM((_NB, _R, z), jnp.float32),
        ] + [pltpu.SemaphoreType.DMA((_NB,)), pltpu.SemaphoreType.DMA((_NB,))],
    )(coef, x)


@jax.jit
def kernel(x, mean_list, scale_list, weight_logits):
    scale = jax.nn.softplus(scale_list)
    ninv = -0.5 / (scale * scale)
    log_w = jax.nn.log_softmax(weight_logits, axis=-1)
    cns = -jnp.log(scale) - 0.5 * np.log(2.0 * np.pi) + log_w[:, None, :]
    coef = jnp.concatenate(
        [mean_list[0].T, ninv[0].T, cns[0].T], axis=0)
    return _sc_call(coef, x)


# speedup vs baseline: 1.0664x; 1.0166x over previous
import jax
import jax.numpy as jnp
import numpy as np
from jax import lax
from jax.experimental import pallas as pl
from jax.experimental.pallas import tpu as pltpu
from jax.experimental.pallas import tpu_sc as plsc

_K = 8

_NC = 2
_NS = 16
_NW = _NC * _NS
_L = 16
_R = 16
_NB = 2

_LOG2P = (
    0.0008254628229340533, 1.415653190432736, -0.5687040530057521,
    0.15270028479752185,
)
_LN2 = 0.6931471805599453


def _vlog2(y):
    bits = lax.bitcast_convert_type(y, jnp.int32)
    e = (bits >> 23) - 127
    f = lax.bitcast_convert_type((bits & 0x007FFFFF) | 0x3F800000, jnp.float32)
    t = f - 1.0
    p = _LOG2P[-1] * t + _LOG2P[-2]
    for c in _LOG2P[-3::-1]:
        p = p * t + c
    return e.astype(jnp.float32) + p


def _mix_logprob(xv, mk, nk, ck):
    ls = [(xv - mk[k]) * (xv - mk[k]) * nk[k] + ck[k] for k in range(_K)]
    m0 = jnp.maximum(jnp.maximum(ls[0], ls[1]), jnp.maximum(ls[2], ls[3]))
    m1 = jnp.maximum(jnp.maximum(ls[4], ls[5]), jnp.maximum(ls[6], ls[7]))
    lmax = jnp.maximum(m0, m1)
    s = None
    for k in range(_K):
        e = jnp.exp(ls[k] - lmax)
        s = e if s is None else s + e
    return lmax, s


def _sc_body(coef_hbm, x_hbm, o_hbm, coef_v, buf, isem, osem):
    wid = lax.axis_index("s") * _NC + lax.axis_index("c")
    rows_per = x_hbm.shape[0] // _NW
    z = x_hbm.shape[1]
    nchunk = rows_per // _R
    base = wid * rows_per

    def in_cp(c):
        p = lax.bitwise_and(c, _NB - 1)
        return pltpu.make_async_copy(
            x_hbm.at[pl.ds(base + c * _R, _R)], buf.at[p], isem.at[p])

    def out_cp(c):
        p = lax.bitwise_and(c, _NB - 1)
        return pltpu.make_async_copy(
            buf.at[p], o_hbm.at[pl.ds(base + c * _R, _R)], osem.at[p])

    in_cp(0).start()
    pltpu.sync_copy(coef_hbm, coef_v)

    def chunk_body(c, carry):
        p = lax.bitwise_and(c, _NB - 1)

        in_cp(c).wait()

        def z_body(zb, c2):
            @pl.when(jnp.logical_and(zb == 16, c >= _NB - 1))
            def _():
                out_cp(c - (_NB - 1)).wait()

            @pl.when(jnp.logical_and(zb == 16, c + 1 < nchunk))
            def _():
                in_cp(c + 1).start()

            zsl = pl.ds(zb * _L, _L)
            mk = [coef_v[k, zsl] for k in range(_K)]
            nk = [coef_v[_K + k, zsl] for k in range(_K)]
            ck = [coef_v[2 * _K + k, zsl] for k in range(_K)]
            for r in range(_R):
                lmax, s = _mix_logprob(buf[p, r, zsl], mk, nk, ck)
                buf[p, r, zsl] = lmax + _vlog2(s) * _LN2
            return c2

        lax.fori_loop(0, z // _L, z_body, 0)
        out_cp(c).start()
        return carry

    lax.fori_loop(0, nchunk, chunk_body, 0)
    for c in range(nchunk - _NB + 1, nchunk):
        out_cp(jnp.int32(c)).wait()


def _sc_call(coef, x):
    b, z = x.shape
    mesh = plsc.VectorSubcoreMesh(core_axis_name="c", subcore_axis_name="s")
    return pl.kernel(
        _sc_body,
        mesh=mesh,
        out_type=jax.ShapeDtypeStruct((b, z), jnp.float32),
        scratch_types=[
            pltpu.VMEM((3 * _K, z), jnp.float32),
            pltpu.VMEM((_NB, _R, z), jnp.float32),
        ] + [pltpu.SemaphoreType.DMA((_NB,)), pltpu.SemaphoreType.DMA((_NB,))],
    )(coef, x)


@jax.jit
def kernel(x, mean_list, scale_list, weight_logits):
    scale = jax.nn.softplus(scale_list)
    ninv = -0.5 / (scale * scale)
    log_w = jax.nn.log_softmax(weight_logits, axis=-1)
    cns = -jnp.log(scale) - 0.5 * np.log(2.0 * np.pi) + log_w[:, None, :]
    coef = jnp.concatenate(
        [mean_list[0].T, ninv[0].T, cns[0].T], axis=0)
    return _sc_call(coef, x)
